# baseline (speedup 1.0000x reference)
import functools

import jax
import jax.numpy as jnp
from jax.experimental import pallas as pl
from jax.experimental.pallas import tpu as pltpu

_P_LAMBDA = 0.4
_K = 50


def _fused_body(n, ni, npad, vec_ref, sk_ref, skr_ref, wm1_ref, bm1_ref,
                wm2_ref, bm2_ref, wa_ref, wb_ref, wc_ref, bp1_ref, wp2_ref,
                bp2_ref, out_ref, sc_ref):
    vec = vec_ref[...]
    t = vec.shape[0]

    hm = jnp.maximum(
        jnp.dot(vec, wm1_ref[...], preferred_element_type=jnp.float32)
        + bm1_ref[...], 0.0)
    ms_c = (jnp.dot(hm, wm2_ref[...], preferred_element_type=jnp.float32)
            + bm2_ref[...])
    ms_r = jnp.transpose(ms_c)

    ic = jax.lax.broadcasted_iota(jnp.int32, (t, t), 0)
    ir = jax.lax.broadcasted_iota(jnp.int32, (t, t), 1)
    npre = jnp.where((ms_c > ms_r) | ((ms_c == ms_r) & (ic < ir)), 1.0, 0.0)
    rank1_r = jnp.sum(npre, axis=0, keepdims=True)
    rank1_c = (t - 1) - jnp.sum(npre, axis=1, keepdims=True)
    kept_c = rank1_c < n
    kept_r = rank1_r < n
    sk_c = sk_ref[...]
    sk_r = skr_ref[...]
    q = jnp.where(
        kept_c & ((sk_c > sk_r) | ((sk_c == sk_r) & (rank1_c < rank1_r))),
        1.0, 0.0)
    rank2_r = jnp.sum(q, axis=0, keepdims=True)
    pos = jnp.where(kept_r, rank2_r, float(npad - 1)).astype(jnp.int32)
    ip = jax.lax.broadcasted_iota(jnp.int32, (npad, t), 0)
    onehot = jnp.where(ip == pos, 1.0, 0.0)

    oneb = onehot.astype(jnp.bfloat16)
    vhi = vec.astype(jnp.bfloat16)
    vlo = (vec - vhi.astype(jnp.float32)).astype(jnp.bfloat16)
    vp = (jnp.dot(oneb, vhi, preferred_element_type=jnp.float32)
          + jnp.dot(oneb, vlo, preferred_element_type=jnp.float32))
    sp = jnp.sum(onehot * ms_r, axis=1, keepdims=True)

    a = jnp.dot(vp, wa_ref[...], preferred_element_type=jnp.float32)
    b = jnp.dot(vp, wb_ref[...], preferred_element_type=jnp.float32)
    ai = a[:ni] + bp1_ref[...]
    vi = vp[:ni]
    base = sp[:ni] + bp2_ref[0, 0]
    wc = wc_ref[...]
    wp2r = jnp.transpose(wp2_ref[...])
    row = jax.lax.broadcasted_iota(jnp.int32, (ni, 1), 0)
    nsh = ni + (_K // 8) * 8
    vsh = [vp[r:r + nsh] for r in range(8)]
    bsh = [b[r:r + nsh] for r in range(8)]
    ssh = [sp[r:r + nsh] for r in range(8)]
    for k in range(1, _K + 1):
        kq, kr = (k // 8) * 8, k % 8
        vj = vsh[kr][kq:kq + ni]
        bj = bsh[kr][kq:kq + ni]
        sj = ssh[kr][kq:kq + ni]
        h = jnp.maximum(
            ai + bj + jnp.dot(vi * vj, wc, preferred_element_type=jnp.float32),
            0.0)
        ps = jnp.sum(h * wp2r, axis=1, keepdims=True)
        col = ps + sj + base
        col = jnp.where(row < n - k, col, -1e9)
        sc_ref[:, k - 1:k] = col

    sc_ref[:, _K:_K + 1] = jnp.zeros((ni, 1), jnp.float32)
    sc = sc_ref[...]
    mx = jnp.max(sc, axis=1, keepdims=True)
    e = jnp.exp(sc - mx)
    out_ref[...] = (e / jnp.sum(e, axis=1, keepdims=True))[:n - 1]


def kernel(vectors, span_starts, span_ends, Wm1, bm1, Wm2, bm2, Wp1, bp1,
           Wp2, bp2):
    t, d = vectors.shape
    n = int(_P_LAMBDA * t)
    hidden = Wm1.shape[1]
    ni = ((n - 1) + 7) // 8 * 8
    npad = 1024
    skey = (span_starts * 100000 + span_ends).astype(jnp.int32)
    wa, wb, wc = Wp1[:d], Wp1[d:2 * d], Wp1[2 * d:]

    return pl.pallas_call(
        functools.partial(_fused_body, n, ni, npad),
        out_shape=jax.ShapeDtypeStruct((n - 1, _K + 1), jnp.float32),
        scratch_shapes=[pltpu.VMEM((ni, _K + 1), jnp.float32)],
    )(vectors, skey.reshape(t, 1), skey.reshape(1, t),
      Wm1, bm1.reshape(1, hidden), Wm2, bm2.reshape(1, 1),
      wa, wb, wc, bp1.reshape(1, hidden), Wp2, bp2.reshape(1, 1))


# speedup vs baseline: 2.0343x; 2.0343x over previous
import functools

import jax
import jax.numpy as jnp
from jax.experimental import pallas as pl
from jax.experimental.pallas import tpu as pltpu

_P_LAMBDA = 0.4
_K = 50


def _fused_body(n, ni, npad, vec_ref, ss_ref, se_ref, wm1_ref, bm1_ref,
                wm2_ref, bm2_ref, wp1_ref, bp1_ref, wp2_ref,
                bp2_ref, out_ref, sc_ref):
    vec = vec_ref[...]
    t, d = vec.shape

    hm = jnp.maximum(
        jnp.dot(vec, wm1_ref[...], preferred_element_type=jnp.float32)
        + bm1_ref[...], 0.0)
    ms_c = (jnp.dot(hm, wm2_ref[...], preferred_element_type=jnp.float32)
            + bm2_ref[...])
    ms_r = jnp.transpose(ms_c)

    ic = jax.lax.broadcasted_iota(jnp.int32, (t, t), 0)
    ir = jax.lax.broadcasted_iota(jnp.int32, (t, t), 1)
    npre = jnp.where((ms_c > ms_r) | ((ms_c == ms_r) & (ic < ir)), 1.0, 0.0)
    rank1_r = jnp.sum(npre, axis=0, keepdims=True)
    rank1_c = (t - 1) - jnp.sum(npre, axis=1, keepdims=True)
    kept_c = rank1_c < n
    kept_r = rank1_r < n
    sk_r = ss_ref[...] * 100000 + se_ref[...]
    sk_c = jnp.transpose(sk_r)
    q = jnp.where(
        kept_c & ((sk_c > sk_r) | ((sk_c == sk_r) & (rank1_c < rank1_r))),
        1.0, 0.0)
    rank2_r = jnp.sum(q, axis=0, keepdims=True)
    pos = jnp.where(kept_r, rank2_r, float(npad - 1)).astype(jnp.int32)
    ip = jax.lax.broadcasted_iota(jnp.int32, (npad, t), 0)
    onehot = jnp.where(ip == pos, 1.0, 0.0)

    oneb = onehot.astype(jnp.bfloat16)
    vhi = vec.astype(jnp.bfloat16)
    vlo = (vec - vhi.astype(jnp.float32)).astype(jnp.bfloat16)
    vp = (jnp.dot(oneb, vhi, preferred_element_type=jnp.float32)
          + jnp.dot(oneb, vlo, preferred_element_type=jnp.float32))
    sp = jnp.sum(onehot * ms_r, axis=1, keepdims=True)

    a = jnp.dot(vp, wp1_ref[0:d], preferred_element_type=jnp.float32)
    b = jnp.dot(vp, wp1_ref[d:2 * d], preferred_element_type=jnp.float32)
    ai = a[:ni] + bp1_ref[...]
    vi = vp[:ni]
    base = sp[:ni] + bp2_ref[0, 0]
    wc = wp1_ref[2 * d:3 * d]
    wp2 = wp2_ref[...]
    row = jax.lax.broadcasted_iota(jnp.int32, (ni, 1), 0)
    nsh = ni + (_K // 8) * 8
    vsh = [vp[r:r + nsh] for r in range(8)]
    bsh = [b[r:r + nsh] for r in range(8)]
    ssh = [sp[r:r + nsh] for r in range(8)]
    for k in range(1, _K + 1):
        kq, kr = (k // 8) * 8, k % 8
        vj = vsh[kr][kq:kq + ni]
        bj = bsh[kr][kq:kq + ni]
        sj = ssh[kr][kq:kq + ni]
        h = jnp.maximum(
            ai + bj + jnp.dot(vi * vj, wc, preferred_element_type=jnp.float32),
            0.0)
        ps = jnp.dot(h, wp2, preferred_element_type=jnp.float32)
        col = ps + sj + base
        col = jnp.where(row < n - k, col, -1e9)
        sc_ref[:, k - 1:k] = col

    sc_ref[:, _K:_K + 1] = jnp.zeros((ni, 1), jnp.float32)
    sc = sc_ref[...]
    mx = jnp.max(sc, axis=1, keepdims=True)
    e = jnp.exp(sc - mx)
    out_ref[...] = (e / jnp.sum(e, axis=1, keepdims=True))[:n - 1]


def kernel(vectors, span_starts, span_ends, Wm1, bm1, Wm2, bm2, Wp1, bp1,
           Wp2, bp2):
    t, d = vectors.shape
    n = int(_P_LAMBDA * t)
    hidden = Wm1.shape[1]
    ni = ((n - 1) + 7) // 8 * 8
    npad = 1024

    return pl.pallas_call(
        functools.partial(_fused_body, n, ni, npad),
        out_shape=jax.ShapeDtypeStruct((n - 1, _K + 1), jnp.float32),
        scratch_shapes=[pltpu.VMEM((ni, _K + 1), jnp.float32)],
    )(vectors, span_starts.astype(jnp.int32).reshape(1, t),
      span_ends.astype(jnp.int32).reshape(1, t),
      Wm1, bm1.reshape(1, hidden), Wm2, bm2.reshape(1, 1),
      Wp1, bp1.reshape(1, hidden), Wp2, bp2.reshape(1, 1))
